# Initial kernel scaffold; baseline (speedup 1.0000x reference)
import jax
import jax.numpy as jnp
from jax.experimental import pallas as pl


def kernel(input_ids, word_table, pos_table, scale, shift):
    raise NotImplementedError("write your pallas kernel here")


# speedup vs baseline: 1.3851x; 1.3851x over previous
import functools

import jax
import jax.numpy as jnp
from jax import lax
from jax.experimental import pallas as pl
from jax.experimental.pallas import tpu as pltpu
from jax.experimental.pallas import tpu_sc as plsc

D = 64
SEQ = 200
L = 16
NV = D // L
NC = 2
NS = 16
NW = NC * NS


def _hsum16(v):
    lanes = lax.iota(jnp.int32, L)
    dnums = lax.GatherDimensionNumbers(
        offset_dims=(), collapsed_slice_dims=(0,), start_index_map=(0,))
    for sh in (8, 4, 2, 1):
        perm = lax.bitwise_xor(lanes, jnp.int32(sh))
        v = v + lax.gather(v, perm[:, None], dnums, slice_sizes=(1,),
                           mode=lax.GatherScatterMode.PROMISE_IN_BOUNDS)
    return v


def _rsqrt16(v):
    i = lax.bitcast_convert_type(v, jnp.int32)
    i = jnp.int32(0x5F3759DF) - lax.shift_right_logical(i, 1)
    y = lax.bitcast_convert_type(i, jnp.float32)
    for _ in range(3):
        y = y * (1.5 - 0.5 * v * y * y)
    return y


@functools.lru_cache(maxsize=None)
def _make_sc_embed(n_rows):
    rows_per_w = n_rows // NW
    seq_per_w = rows_per_w // SEQ
    mesh = plsc.VectorSubcoreMesh(core_axis_name="c", subcore_axis_name="s")

    @functools.partial(
        pl.kernel,
        out_type=jax.ShapeDtypeStruct((n_rows, D), jnp.float32),
        mesh=mesh,
        compiler_params=pltpu.CompilerParams(use_tc_tiling_on_sc=False),
        scratch_types=[
            pltpu.VMEM((rows_per_w,), jnp.int32),
            pltpu.VMEM((SEQ, D), jnp.float32),
            pltpu.VMEM((D,), jnp.float32),
            pltpu.VMEM((D,), jnp.float32),
            pltpu.VMEM((SEQ, D), jnp.float32),
            pltpu.SemaphoreType.DMA,
        ],
    )
    def sc_embed(idx_hbm, wt_hbm, pos_hbm, sc_hbm, sh_hbm, out_hbm,
                 idx_v, pos_v, scale_v, shift_v, rows_v, sem):
        wid = lax.axis_index("s") * NC + lax.axis_index("c")
        row0 = wid * rows_per_w
        pltpu.sync_copy(idx_hbm.at[pl.ds(row0, rows_per_w)], idx_v)
        pltpu.sync_copy(pos_hbm.at[pl.ds(0, SEQ)], pos_v)
        pltpu.sync_copy(sc_hbm, scale_v)
        pltpu.sync_copy(sh_hbm, shift_v)

        scs = [scale_v[pl.ds(q * L, L)] for q in range(NV)]
        shs = [shift_v[pl.ds(q * L, L)] for q in range(NV)]

        def chunk(t, carry):
            pltpu.async_copy(
                wt_hbm.at[idx_v.at[pl.ds(t * SEQ, SEQ)]], rows_v, sem
            ).wait()

            def row(j, c2):
                e = [rows_v[j, pl.ds(q * L, L)] + pos_v[j, pl.ds(q * L, L)]
                     for q in range(NV)]
                mu = _hsum16(e[0] + e[1] + e[2] + e[3]) * (1.0 / D)
                dd = [ei - mu for ei in e]
                q2 = dd[0] * dd[0] + dd[1] * dd[1] + dd[2] * dd[2] + dd[3] * dd[3]
                var = _hsum16(q2) * (1.0 / D)
                y = _rsqrt16(var + 1e-12)
                for q in range(NV):
                    rows_v[j, pl.ds(q * L, L)] = dd[q] * y * scs[q] + shs[q]
                return c2

            lax.fori_loop(0, SEQ, row, 0)
            pltpu.sync_copy(rows_v, out_hbm.at[pl.ds(row0 + t * SEQ, SEQ)])
            return carry

        lax.fori_loop(0, seq_per_w, chunk, 0)

    return sc_embed


def kernel(input_ids, word_table, pos_table, scale, shift):
    B, S = input_ids.shape
    assert S == SEQ and word_table.shape[1] == D
    idx = input_ids.reshape(-1).astype(jnp.int32)
    out = _make_sc_embed(B * S)(idx, word_table, pos_table, scale, shift)
    return out.reshape(B, S, D)


# speedup vs baseline: 2.1750x; 1.5703x over previous
import functools

import jax
import jax.numpy as jnp
from jax import lax
from jax.experimental import pallas as pl
from jax.experimental.pallas import tpu as pltpu
from jax.experimental.pallas import tpu_sc as plsc

D = 64
SEQ = 200
L = 16
NV = D // L
NC = 2
NS = 16
NW = NC * NS


def _hsum16(v):
    lanes = lax.iota(jnp.int32, L)
    dnums = lax.GatherDimensionNumbers(
        offset_dims=(), collapsed_slice_dims=(0,), start_index_map=(0,))
    for sh in (8, 4, 2, 1):
        perm = lax.bitwise_xor(lanes, jnp.int32(sh))
        v = v + lax.gather(v, perm[:, None], dnums, slice_sizes=(1,),
                           mode=lax.GatherScatterMode.PROMISE_IN_BOUNDS)
    return v


def _rsqrt16(v):
    i = lax.bitcast_convert_type(v, jnp.int32)
    i = jnp.int32(0x5F3759DF) - lax.shift_right_logical(i, 1)
    y = lax.bitcast_convert_type(i, jnp.float32)
    for _ in range(2):
        y = y * (1.5 - 0.5 * v * y * y)
    return y


def _ln_row(rows_v, pos_v, j, scs, shs):
    e = [rows_v[j, pl.ds(q * L, L)] + pos_v[j, pl.ds(q * L, L)]
         for q in range(NV)]
    s = (e[0] + e[1]) + (e[2] + e[3])
    q2 = ((e[0] * e[0] + e[1] * e[1]) + (e[2] * e[2] + e[3] * e[3]))
    mu = _hsum16(s) * (1.0 / D)
    exx = _hsum16(q2) * (1.0 / D)
    y = _rsqrt16(exx - mu * mu + 1e-12)
    for q in range(NV):
        rows_v[j, pl.ds(q * L, L)] = ((e[q] - mu) * y) * scs[q] + shs[q]


@functools.lru_cache(maxsize=None)
def _make_sc_embed(n_rows):
    rows_per_w = n_rows // NW
    seq_per_w = rows_per_w // SEQ
    mesh = plsc.VectorSubcoreMesh(core_axis_name="c", subcore_axis_name="s")

    @functools.partial(
        pl.kernel,
        out_type=jax.ShapeDtypeStruct((n_rows, D), jnp.float32),
        mesh=mesh,
        compiler_params=pltpu.CompilerParams(use_tc_tiling_on_sc=False),
        scratch_types=[
            pltpu.VMEM((rows_per_w,), jnp.int32),
            pltpu.VMEM((SEQ, D), jnp.float32),
            pltpu.VMEM((D,), jnp.float32),
            pltpu.VMEM((D,), jnp.float32),
            pltpu.VMEM((SEQ, D), jnp.float32),
            pltpu.SemaphoreType.DMA,
        ],
    )
    def sc_embed(idx_hbm, wt_hbm, pos_hbm, sc_hbm, sh_hbm, out_hbm,
                 idx_v, pos_v, scale_v, shift_v, rows_v, sem):
        wid = lax.axis_index("s") * NC + lax.axis_index("c")
        row0 = wid * rows_per_w
        pltpu.sync_copy(idx_hbm.at[pl.ds(row0, rows_per_w)], idx_v)
        pltpu.sync_copy(pos_hbm.at[pl.ds(0, SEQ)], pos_v)
        pltpu.sync_copy(sc_hbm, scale_v)
        pltpu.sync_copy(sh_hbm, shift_v)

        scs = [scale_v[pl.ds(q * L, L)] for q in range(NV)]
        shs = [shift_v[pl.ds(q * L, L)] for q in range(NV)]

        def chunk(t, carry):
            pltpu.async_copy(
                wt_hbm.at[idx_v.at[pl.ds(t * SEQ, SEQ)]], rows_v, sem
            ).wait()

            def row4(j, c2):
                for u in range(4):
                    _ln_row(rows_v, pos_v, 4 * j + u, scs, shs)
                return c2

            lax.fori_loop(0, SEQ // 4, row4, 0)
            pltpu.sync_copy(rows_v, out_hbm.at[pl.ds(row0 + t * SEQ, SEQ)])
            return carry

        lax.fori_loop(0, seq_per_w, chunk, 0)

    return sc_embed


def kernel(input_ids, word_table, pos_table, scale, shift):
    B, S = input_ids.shape
    assert S == SEQ and word_table.shape[1] == D
    idx = input_ids.reshape(-1).astype(jnp.int32)
    out = _make_sc_embed(B * S)(idx, word_table, pos_table, scale, shift)
    return out.reshape(B, S, D)


# speedup vs baseline: 2.1835x; 1.0039x over previous
import functools

import jax
import jax.numpy as jnp
from jax import lax
from jax.experimental import pallas as pl
from jax.experimental.pallas import tpu as pltpu
from jax.experimental.pallas import tpu_sc as plsc

D = 64
SEQ = 200
L = 16
NV = D // L
NC = 2
NS = 16
NW = NC * NS


def _hsum16(v):
    lanes = lax.iota(jnp.int32, L)
    dnums = lax.GatherDimensionNumbers(
        offset_dims=(), collapsed_slice_dims=(0,), start_index_map=(0,))
    for sh in (8, 4, 2, 1):
        perm = lax.bitwise_xor(lanes, jnp.int32(sh))
        v = v + lax.gather(v, perm[:, None], dnums, slice_sizes=(1,),
                           mode=lax.GatherScatterMode.PROMISE_IN_BOUNDS)
    return v


def _rsqrt16(v):
    i = lax.bitcast_convert_type(v, jnp.int32)
    i = jnp.int32(0x5F3759DF) - lax.shift_right_logical(i, 1)
    y = lax.bitcast_convert_type(i, jnp.float32)
    for _ in range(2):
        y = y * (1.5 - 0.5 * v * y * y)
    return y


def _ln_row(rows_v, pos_v, j, scs, shs):
    e = [rows_v[j, pl.ds(q * L, L)] + pos_v[j, pl.ds(q * L, L)]
         for q in range(NV)]
    s = (e[0] + e[1]) + (e[2] + e[3])
    q2 = ((e[0] * e[0] + e[1] * e[1]) + (e[2] * e[2] + e[3] * e[3]))
    mu = _hsum16(s) * (1.0 / D)
    exx = _hsum16(q2) * (1.0 / D)
    y = _rsqrt16(exx - mu * mu + 1e-12)
    for q in range(NV):
        rows_v[j, pl.ds(q * L, L)] = ((e[q] - mu) * y) * scs[q] + shs[q]


@functools.lru_cache(maxsize=None)
def _make_sc_embed(n_rows):
    rows_per_w = n_rows // NW
    seq_per_w = rows_per_w // SEQ
    mesh = plsc.VectorSubcoreMesh(core_axis_name="c", subcore_axis_name="s")

    @functools.partial(
        pl.kernel,
        out_type=jax.ShapeDtypeStruct((n_rows, D), jnp.float32),
        mesh=mesh,
        compiler_params=pltpu.CompilerParams(use_tc_tiling_on_sc=False),
        scratch_types=[
            pltpu.VMEM((rows_per_w,), jnp.int32),
            pltpu.VMEM((SEQ, D), jnp.float32),
            pltpu.VMEM((D,), jnp.float32),
            pltpu.VMEM((D,), jnp.float32),
            pltpu.VMEM((SEQ, D), jnp.float32),
            pltpu.SemaphoreType.DMA,
        ],
    )
    def sc_embed(idx_hbm, wt_hbm, pos_hbm, sc_hbm, sh_hbm, out_hbm,
                 idx_v, pos_v, scale_v, shift_v, rows_v, sem):
        wid = lax.axis_index("s") * NC + lax.axis_index("c")
        row0 = wid * rows_per_w
        pltpu.sync_copy(idx_hbm.at[pl.ds(row0, rows_per_w)], idx_v)
        pltpu.sync_copy(pos_hbm.at[pl.ds(0, SEQ)], pos_v)
        pltpu.sync_copy(sc_hbm, scale_v)
        pltpu.sync_copy(sh_hbm, shift_v)

        scs = [scale_v[pl.ds(q * L, L)] for q in range(NV)]
        shs = [shift_v[pl.ds(q * L, L)] for q in range(NV)]

        def chunk(t, carry):
            pltpu.async_copy(
                wt_hbm.at[idx_v.at[pl.ds(t * SEQ, SEQ)]], rows_v, sem
            ).wait()

            @plsc.parallel_loop(0, SEQ, step=1, unroll=8)
            def _rows(j):
                _ln_row(rows_v, pos_v, j, scs, shs)
            pltpu.sync_copy(rows_v, out_hbm.at[pl.ds(row0 + t * SEQ, SEQ)])
            return carry

        lax.fori_loop(0, seq_per_w, chunk, 0)

    return sc_embed


def kernel(input_ids, word_table, pos_table, scale, shift):
    B, S = input_ids.shape
    assert S == SEQ and word_table.shape[1] == D
    idx = input_ids.reshape(-1).astype(jnp.int32)
    out = _make_sc_embed(B * S)(idx, word_table, pos_table, scale, shift)
    return out.reshape(B, S, D)
